# baseline (speedup 1.0000x reference)
import functools

import jax
import jax.numpy as jnp
from jax import lax
from jax.experimental import pallas as pl
from jax.experimental.pallas import tpu as pltpu
from jax.experimental.pallas import tpu_sc as plsc

D = 64
NC = 2
NS = 16
NW = NC * NS
C = 128
SUB = 4
S = C * SUB
NBUF = 2


@functools.lru_cache(maxsize=None)
def _emb_kernel(n_total, vocab):
    n_per_w = n_total // NW
    n_chunks = n_per_w // C
    T = n_per_w // S

    mesh = plsc.VectorSubcoreMesh(core_axis_name="c", subcore_axis_name="s")

    @functools.partial(
        pl.kernel,
        mesh=mesh,
        compiler_params=pltpu.CompilerParams(use_tc_tiling_on_sc=False),
        out_type=jax.ShapeDtypeStruct((n_total, D), jnp.float32),
        scratch_types=[
            pltpu.VMEM((n_chunks, C), jnp.int32),
            pltpu.VMEM((S, D), jnp.float32),
            pltpu.VMEM((S, D), jnp.float32),
            pltpu.SemaphoreType.DMA,
            pltpu.SemaphoreType.DMA,
            pltpu.SemaphoreType.DMA,
            pltpu.SemaphoreType.DMA,
        ],
    )
    def k(table_hbm, idx_hbm, out_hbm, idx_v, buf0, buf1, g0, g1, w0, w1):
        bufs = (buf0, buf1)
        gsems = (g0, g1)
        wsems = (w0, w1)
        wid = lax.axis_index("s") * NC + lax.axis_index("c")
        row_base = wid * n_per_w

        pltpu.sync_copy(idx_hbm.at[wid], idx_v)

        def start_gathers(s_id, b):
            for j in range(SUB):
                pltpu.make_async_copy(
                    table_hbm.at[idx_v.at[s_id * SUB + j]],
                    bufs[b].at[pl.ds(j * C, C)],
                    gsems[b],
                ).start()

        def drain_gather(b):
            pltpu.make_async_copy(
                table_hbm.at[pl.ds(0, S)], bufs[b], gsems[b]
            ).wait()

        def start_write(s_id, b):
            pltpu.make_async_copy(
                bufs[b],
                out_hbm.at[pl.ds(row_base + s_id * S, S)],
                wsems[b],
            ).start()

        def drain_write(b):
            pltpu.make_async_copy(
                bufs[b], out_hbm.at[pl.ds(0, S)], wsems[b]
            ).wait()

        for b in range(NBUF):
            start_gathers(b, b)

        def body(t, carry):
            for b in range(NBUF):
                s_id = t * NBUF + b
                drain_gather(b)
                start_write(s_id, b)
                drain_write(b)
                start_gathers(s_id + NBUF, b)
            return carry

        lax.fori_loop(0, T // NBUF - 1, body, 0)

        for b in range(NBUF):
            drain_gather(b)
            start_write(T - NBUF + b, b)
        for b in range(NBUF):
            drain_write(b)

    return k


@jax.jit
def kernel(inputs, table):
    batch, hist = inputs.shape
    n_total = batch * hist
    idx = inputs.reshape(NW, n_total // (NW * C), C)
    out = _emb_kernel(n_total, table.shape[0])(table, idx)
    return out.reshape(batch, hist, D)


# speedup vs baseline: 1.0399x; 1.0399x over previous
import functools

import jax
import jax.numpy as jnp
from jax import lax
from jax.experimental import pallas as pl
from jax.experimental.pallas import tpu as pltpu
from jax.experimental.pallas import tpu_sc as plsc

D = 64
NC = 2
NS = 16
NW = NC * NS
C = 128
SUB = 4
S = C * SUB
NBUF = 2


@functools.lru_cache(maxsize=None)
def _emb_kernel(n_total, vocab):
    n_per_w = n_total // NW
    n_chunks = n_per_w // C
    T = n_per_w // S

    mesh = plsc.VectorSubcoreMesh(core_axis_name="c", subcore_axis_name="s")

    @functools.partial(
        pl.kernel,
        mesh=mesh,
        compiler_params=pltpu.CompilerParams(use_tc_tiling_on_sc=False),
        out_type=jax.ShapeDtypeStruct((n_total, D), jnp.float32),
        scratch_types=[
            pltpu.VMEM((n_chunks, C), jnp.int32),
            pltpu.VMEM((S, D), jnp.float32),
            pltpu.VMEM((S, D), jnp.float32),
            pltpu.SemaphoreType.DMA,
            pltpu.SemaphoreType.DMA,
            pltpu.SemaphoreType.DMA,
            pltpu.SemaphoreType.DMA,
        ],
    )
    def k(table_hbm, idx_hbm, out_hbm, idx_v, buf0, buf1, g0, g1, w0, w1):
        bufs = (buf0, buf1)
        gsems = (g0, g1)
        wsems = (w0, w1)
        wid = lax.axis_index("s") * NC + lax.axis_index("c")
        row_base = wid * n_per_w

        pltpu.sync_copy(idx_hbm.at[wid], idx_v)

        def start_gathers(s_id, b):
            for j in range(SUB):
                pltpu.make_async_copy(
                    table_hbm.at[idx_v.at[s_id * SUB + j]],
                    bufs[b].at[pl.ds(j * C, C)],
                    gsems[b],
                ).start()

        def drain_gather(b):
            pltpu.make_async_copy(
                table_hbm.at[pl.ds(0, S)], bufs[b], gsems[b]
            ).wait()

        def start_write(s_id, b):
            pltpu.make_async_copy(
                bufs[b],
                out_hbm.at[pl.ds(row_base + s_id * S, S)],
                wsems[b],
            ).start()

        def drain_write(b):
            pltpu.make_async_copy(
                bufs[b], out_hbm.at[pl.ds(0, S)], wsems[b]
            ).wait()

        for b in range(NBUF):
            start_gathers(b, b)

        def body(t, carry):
            for b in range(NBUF):
                s_id = t * NBUF + b
                drain_gather(b)
                start_write(s_id, b)
                drain_write(b)
                start_gathers(s_id + NBUF, b)
            return carry

        lax.fori_loop(0, T // NBUF - 1, body, 0)

        for b in range(NBUF):
            drain_gather(b)
            start_write(T - NBUF + b, b)
        for b in range(NBUF):
            drain_write(b)

    return k


@jax.jit
def kernel(inputs, table):
    batch, hist = inputs.shape
    n_total = batch * hist
    idx = inputs.T.reshape(NW, n_total // (NW * C), C)
    out = _emb_kernel(n_total, table.shape[0])(table, idx)
    return out.reshape(hist, batch, D).transpose(1, 0, 2)


# speedup vs baseline: 1.2026x; 1.1564x over previous
import functools

import jax
import jax.numpy as jnp
from jax import lax
from jax.experimental import pallas as pl
from jax.experimental.pallas import tpu as pltpu
from jax.experimental.pallas import tpu_sc as plsc

D = 64
DP = 128
NC = 2
NS = 16
NW = NC * NS
C = 128
S = 256
SUB = S // C
NBUF = 2


@functools.lru_cache(maxsize=None)
def _emb_kernel(batch, hist, vocab):
    bw = batch // NW
    nch = bw // C
    nhalf = bw // S
    T = hist * nhalf

    mesh = plsc.VectorSubcoreMesh(core_axis_name="c", subcore_axis_name="s")

    @functools.partial(
        pl.kernel,
        mesh=mesh,
        compiler_params=pltpu.CompilerParams(use_tc_tiling_on_sc=True),
        out_type=jax.ShapeDtypeStruct((batch, hist, DP), jnp.float32),
        scratch_types=[
            pltpu.VMEM((hist, nch, C), jnp.int32),
            pltpu.VMEM((S, DP), jnp.float32),
            pltpu.VMEM((S, DP), jnp.float32),
            pltpu.SemaphoreType.DMA,
            pltpu.SemaphoreType.DMA,
            pltpu.SemaphoreType.DMA,
            pltpu.SemaphoreType.DMA,
        ],
    )
    def k(table_hbm, idx_hbm, out_hbm, idx_v, buf0, buf1, g0, g1, w0, w1):
        bufs = (buf0, buf1)
        gsems = (g0, g1)
        wsems = (w0, w1)
        wid = lax.axis_index("s") * NC + lax.axis_index("c")
        b0 = wid * bw

        pltpu.sync_copy(idx_hbm.at[:, pl.ds(wid * nch, nch)], idx_v)

        def start_gathers(s_id, b):
            h = s_id // nhalf
            half = s_id % nhalf
            for j in range(SUB):
                pltpu.make_async_copy(
                    table_hbm.at[idx_v.at[h, half * SUB + j]],
                    bufs[b].at[pl.ds(j * C, C)],
                    gsems[b],
                ).start()

        def drain_gather(b):
            pltpu.make_async_copy(
                table_hbm.at[pl.ds(0, S)], bufs[b], gsems[b]
            ).wait()

        def start_write(s_id, b):
            h = s_id // nhalf
            half = s_id % nhalf
            pltpu.make_async_copy(
                bufs[b],
                out_hbm.at[pl.ds(b0 + half * S, S), h],
                wsems[b],
            ).start()

        def drain_write(b):
            pltpu.make_async_copy(
                bufs[b], out_hbm.at[pl.ds(0, S), 0], wsems[b]
            ).wait()

        for b in range(NBUF):
            start_gathers(b, b)

        def body(t, carry):
            for b in range(NBUF):
                s_id = t * NBUF + b
                drain_gather(b)
                start_write(s_id, b)
                drain_write(b)
                start_gathers(s_id + NBUF, b)
            return carry

        lax.fori_loop(0, T // NBUF - 1, body, 0)

        for b in range(NBUF):
            drain_gather(b)
            start_write(T - NBUF + b, b)
        for b in range(NBUF):
            drain_write(b)

    return k


@jax.jit
def kernel(inputs, table):
    batch, hist = inputs.shape
    table_p = jnp.pad(table, ((0, 0), (0, DP - D)))
    idx = inputs.T.reshape(hist, batch // C, C)
    out_p = _emb_kernel(batch, hist, table.shape[0])(table_p, idx)
    return out_p[:, :, :D]
